# Initial kernel scaffold; baseline (speedup 1.0000x reference)
import functools

import jax
import jax.numpy as jnp
from jax import lax
from jax.experimental import pallas as pl
from jax.experimental.pallas import tpu as pltpu
from jax.experimental.pallas import tpu_sc as plsc

_NC = 2
_NS = 16
_K = 100
_INNER = 20


def _dotT(a, b):
    return lax.dot_general(a, b, (((1,), (1,)), ((), ())),
                           precision=lax.Precision.HIGHEST,
                           preferred_element_type=jnp.float32)


@functools.lru_cache(maxsize=None)
def _sc_accumulate(n, e, d, de1):
    nw = _NC * _NS
    ew = e // nw
    rows_per_tile = n // _NS
    outer = ew // (_INNER * _K)
    idx_rows_per_worker = ew // _K

    mesh = plsc.VectorSubcoreMesh(core_axis_name="c", subcore_axis_name="s")

    @functools.partial(
        pl.kernel,
        mesh=mesh,
        out_type=[
            jax.ShapeDtypeStruct((_NC, n, d), jnp.float32),
            jax.ShapeDtypeStruct((_NC, n, de1), jnp.float32),
            jax.ShapeDtypeStruct((_NC, n, de1), jnp.float32),
        ],
        scratch_types=[
            pltpu.VMEM((_K, d), jnp.float32),
            pltpu.VMEM((_INNER, _K), jnp.int32),
            pltpu.VMEM((_INNER, _K), jnp.int32),
            pltpu.VMEM((_INNER * _K, de1), jnp.float32),
            pltpu.VMEM_SHARED((n, d), jnp.float32),
            pltpu.VMEM_SHARED((n, de1), jnp.float32),
            pltpu.VMEM_SHARED((n, de1), jnp.float32),
            pltpu.SemaphoreType.DMA,
        ],
    )
    def sc_kernel(atom_hbm, sidx_hbm, nidx_hbm, d17_hbm, zd_hbm, zde1_hbm,
                  g2_out, dc2_out, rc2_out,
                  rows_v, sidx_v, nidx_v, d17_v, gacc, dacc, racc, sem):
        cid = lax.axis_index("c")
        sid = lax.axis_index("s")
        wid = sid * _NC + cid

        r0 = sid * rows_per_tile
        pltpu.sync_copy(zd_hbm.at[pl.ds(r0, rows_per_tile), :],
                        gacc.at[pl.ds(r0, rows_per_tile), :])
        pltpu.sync_copy(zde1_hbm.at[pl.ds(r0, rows_per_tile), :],
                        dacc.at[pl.ds(r0, rows_per_tile), :])
        pltpu.sync_copy(zde1_hbm.at[pl.ds(r0, rows_per_tile), :],
                        racc.at[pl.ds(r0, rows_per_tile), :])
        plsc.subcore_barrier()

        base_row = wid * idx_rows_per_worker

        def body(o, carry):
            row0 = base_row + o * _INNER
            pltpu.sync_copy(sidx_hbm.at[pl.ds(row0, _INNER), :], sidx_v)
            pltpu.sync_copy(nidx_hbm.at[pl.ds(row0, _INNER), :], nidx_v)
            pltpu.sync_copy(d17_hbm.at[pl.ds(row0 * _K, _INNER * _K), :],
                            d17_v)
            for j in range(_INNER):
                pltpu.async_copy(atom_hbm.at[nidx_v.at[j]], rows_v,
                                 sem).wait()
                pltpu.sync_copy(rows_v, gacc.at[sidx_v.at[j]], add=True)
                pltpu.sync_copy(d17_v.at[pl.ds(j * _K, _K), :],
                                dacc.at[sidx_v.at[j]], add=True)
                pltpu.sync_copy(d17_v.at[pl.ds(j * _K, _K), :],
                                racc.at[nidx_v.at[j]], add=True)
            return carry

        lax.fori_loop(0, outer, body, 0)
        plsc.subcore_barrier()

        pltpu.sync_copy(gacc.at[pl.ds(r0, rows_per_tile), :],
                        g2_out.at[cid, pl.ds(r0, rows_per_tile), :])
        pltpu.sync_copy(dacc.at[pl.ds(r0, rows_per_tile), :],
                        dc2_out.at[cid, pl.ds(r0, rows_per_tile), :])
        pltpu.sync_copy(racc.at[pl.ds(r0, rows_per_tile), :],
                        rc2_out.at[cid, pl.ds(r0, rows_per_tile), :])

    return sc_kernel


def _xtx_body(x_ref, o_ref):
    @pl.when(pl.program_id(0) == 0)
    def _init():
        o_ref[...] = jnp.zeros_like(o_ref)

    x = x_ref[...]
    o_ref[...] += lax.dot_general(x, x, (((0,), (0,)), ((), ())),
                                  precision=lax.Precision.HIGHEST,
                                  preferred_element_type=jnp.float32)


def _xtx(x):
    rows = x.shape[0]
    blk = 4000
    grid = rows // blk
    return pl.pallas_call(
        _xtx_body,
        grid=(grid,),
        in_specs=[pl.BlockSpec((blk, 128), lambda i: (i, 0))],
        out_specs=pl.BlockSpec((128, 128), lambda i: (0, 0)),
        out_shape=jax.ShapeDtypeStruct((128, 128), jnp.float32),
    )(x)


def _finish_body(e_edges, de, g2_ref, dc2_ref, rc2_ref, atom_ref, xtx_ref,
                 w1_ref, b1_ref, g1_ref, be1_ref, w2_ref, b2_ref, g2v_ref,
                 be2_ref, out_ref):
    ef = jnp.float32(e_edges)
    G = g2_ref[0] + g2_ref[1]
    DC = dc2_ref[0] + dc2_ref[1]
    RC = rc2_ref[0] + rc2_ref[1]
    cnt = DC[:, de:de + 1]
    Dsum = DC[:, :de]
    c = RC[:, de:de + 1]
    R = RC[:, :de]
    atom = atom_ref[...]
    W1 = w1_ref[...]
    d = atom.shape[1]
    W1a = W1[:, :d]
    W1b = W1[:, d:]
    b1 = b1_ref[...]
    g1 = g1_ref[...]
    be1 = be1_ref[...]
    b2 = b2_ref[...]
    g2v = g2v_ref[...]
    be2 = be2_ref[...]

    inv = 1.0 / jnp.maximum(cnt, 1.0)
    M1 = _dotT(G * inv, W1a) + _dotT(Dsum * inv, W1b) + b1

    Gs = jnp.sum(G, axis=0, keepdims=True)
    Ds = jnp.sum(Dsum, axis=0, keepdims=True)
    m1 = (_dotT(Gs, W1a) + _dotT(Ds, W1b)) / ef + b1

    z = _dotT(atom, W1a)
    tR = _dotT(R, W1b)
    xtx = xtx_ref[...]
    dd = jnp.zeros((de, de), jnp.float32)
    for i in range(128 // de):
        dd = dd + xtx[i * de:(i + 1) * de, i * de:(i + 1) * de]
    Wdd = jnp.dot(W1b, dd, precision=lax.Precision.HIGHEST,
                  preferred_element_type=jnp.float32)
    tt = jnp.sum(Wdd * W1b, axis=1)[None, :]

    q = (jnp.sum(c * z * z, axis=0, keepdims=True)
         + 2.0 * jnp.sum(z * tR, axis=0, keepdims=True)
         + 2.0 * b1 * jnp.sum(c * z, axis=0, keepdims=True)
         + tt
         + 2.0 * b1 * _dotT(Ds, W1b)
         + ef * b1 * b1)
    v1 = q / ef - m1 * m1
    scale1 = g1 * lax.rsqrt(v1 + 1e-5)
    safe = cnt > 0.5
    fea = jnp.where(safe, (M1 - m1) * scale1 + be1, 0.0)

    node = _dotT(jnp.where(safe, atom, 0.0), w2_ref[...]) + b2
    m2 = jnp.mean(node, axis=0, keepdims=True)
    v2 = jnp.mean(node * node, axis=0, keepdims=True) - m2 * m2
    nb = (node - m2) * (g2v * lax.rsqrt(v2 + 1e-5)) + be2

    x = nb + fea
    out_ref[...] = jnp.maximum(x, 0.0) + jnp.log(1.0 + jnp.exp(-jnp.abs(x)))


def _finish(e_edges, de, g2, dc2, rc2, atom, xtx, W1, b1, g1, be1, W2, b2,
            g2v, be2):
    n, d = atom.shape
    return pl.pallas_call(
        functools.partial(_finish_body, e_edges, de),
        out_shape=jax.ShapeDtypeStruct((n, d), jnp.float32),
    )(g2, dc2, rc2, atom, xtx, W1, b1, g1, be1, W2, b2, g2v, be2)


def kernel(atom_fea, nbr_dist_fea, nbr_adj_value, nbr_bond_type,
           self_fea_idx, nbr_fea_idx, ads_atom_idx,
           W1, b1, g1, be1, W2, b2, g2, be2):
    n, d = atom_fea.shape
    e, de = nbr_dist_fea.shape
    de1 = de + 1

    d17 = jnp.concatenate(
        [nbr_dist_fea, jnp.ones((e, 1), jnp.float32)], axis=1)
    sidx2 = self_fea_idx.reshape(e // _K, _K)
    nidx2 = nbr_fea_idx.reshape(e // _K, _K)
    zd = jnp.zeros((n, d), jnp.float32)
    zde1 = jnp.zeros((n, de1), jnp.float32)

    g2p, dc2p, rc2p = _sc_accumulate(n, e, d, de1)(
        atom_fea, sidx2, nidx2, d17, zd, zde1)

    xtx = _xtx(nbr_dist_fea.reshape(e // 8, 8 * de))

    row = lambda v: v.reshape(1, d)
    return _finish(e, de, g2p, dc2p, rc2p, atom_fea, xtx, W1, row(b1),
                   row(g1), row(be1), W2, row(b2), row(g2), row(be2))


# speedup vs baseline: 4.5806x; 4.5806x over previous
import functools

import jax
import jax.numpy as jnp
from jax import lax
from jax.experimental import pallas as pl
from jax.experimental.pallas import tpu as pltpu
from jax.experimental.pallas import tpu_sc as plsc

_NC = 2
_NS = 16
_K = 100
_INNER = 20


def _pad_n(n):
    return ((n + _NS - 1) // _NS) * _NS


def _dotT(a, b):
    return lax.dot_general(a, b, (((1,), (1,)), ((), ())),
                           precision=lax.Precision.HIGHEST,
                           preferred_element_type=jnp.float32)


@functools.lru_cache(maxsize=None)
def _sc_accumulate(n, e, d, de1):
    nw = _NC * _NS
    ew = e // nw
    npad = _pad_n(n)
    rows_per_tile = npad // _NS
    outer = ew // (_INNER * _K)
    idx_rows_per_worker = ew // _K

    mesh = plsc.VectorSubcoreMesh(core_axis_name="c", subcore_axis_name="s")

    @functools.partial(
        pl.kernel,
        mesh=mesh,
        compiler_params=pltpu.CompilerParams(use_tc_tiling_on_sc=False),
        out_type=[
            jax.ShapeDtypeStruct((_NC, npad, d), jnp.float32),
            jax.ShapeDtypeStruct((_NC, npad, de1), jnp.float32),
            jax.ShapeDtypeStruct((_NC, npad, de1), jnp.float32),
        ],
        scratch_types=[
            pltpu.VMEM((_K, d), jnp.float32),
            pltpu.VMEM((_INNER, _K), jnp.int32),
            pltpu.VMEM((_INNER, _K), jnp.int32),
            pltpu.VMEM((_K, de1), jnp.float32),
            pltpu.VMEM_SHARED((npad, d), jnp.float32),
            pltpu.VMEM_SHARED((npad, de1), jnp.float32),
            pltpu.VMEM_SHARED((npad, de1), jnp.float32),
            pltpu.SemaphoreType.DMA,
        ],
    )
    def sc_kernel(atom_hbm, sidx_hbm, nidx_hbm, d17_hbm, zd_hbm, zde1_hbm,
                  g2_out, dc2_out, rc2_out,
                  rows_v, sidx_v, nidx_v, d17_v, gacc, dacc, racc, sem):
        cid = lax.axis_index("c")
        sid = lax.axis_index("s")
        wid = sid * _NC + cid

        r0 = sid * rows_per_tile
        pltpu.sync_copy(zd_hbm.at[pl.ds(r0, rows_per_tile), :],
                        gacc.at[pl.ds(r0, rows_per_tile), :])
        pltpu.sync_copy(zde1_hbm.at[pl.ds(r0, rows_per_tile), :],
                        dacc.at[pl.ds(r0, rows_per_tile), :])
        pltpu.sync_copy(zde1_hbm.at[pl.ds(r0, rows_per_tile), :],
                        racc.at[pl.ds(r0, rows_per_tile), :])
        plsc.subcore_barrier()

        base_row = wid * idx_rows_per_worker
        edge_base = wid * ew

        def body(o, carry):
            pltpu.sync_copy(sidx_hbm.at[pl.ds(base_row + o * _INNER,
                                              _INNER), :], sidx_v)
            pltpu.sync_copy(nidx_hbm.at[pl.ds(base_row + o * _INNER,
                                              _INNER), :], nidx_v)
            for j in range(_INNER):
                pltpu.sync_copy(
                    d17_hbm.at[pl.ds(edge_base + (o * _INNER + j) * _K,
                                     _K), :], d17_v)
                pltpu.async_copy(atom_hbm.at[nidx_v.at[j]], rows_v,
                                 sem).wait()
                pltpu.sync_copy(rows_v, gacc.at[sidx_v.at[j]], add=True)
                pltpu.sync_copy(d17_v, dacc.at[sidx_v.at[j]], add=True)
                pltpu.sync_copy(d17_v, racc.at[nidx_v.at[j]], add=True)
            return carry

        lax.fori_loop(0, outer, body, 0)
        plsc.subcore_barrier()

        pltpu.sync_copy(gacc.at[pl.ds(r0, rows_per_tile), :],
                        g2_out.at[cid, pl.ds(r0, rows_per_tile), :])
        pltpu.sync_copy(dacc.at[pl.ds(r0, rows_per_tile), :],
                        dc2_out.at[cid, pl.ds(r0, rows_per_tile), :])
        pltpu.sync_copy(racc.at[pl.ds(r0, rows_per_tile), :],
                        rc2_out.at[cid, pl.ds(r0, rows_per_tile), :])

    return sc_kernel


def _xtx_body(x_ref, o_ref):
    @pl.when(pl.program_id(0) == 0)
    def _init():
        o_ref[...] = jnp.zeros_like(o_ref)

    x = x_ref[...]
    o_ref[...] += lax.dot_general(x, x, (((0,), (0,)), ((), ())),
                                  precision=lax.Precision.HIGHEST,
                                  preferred_element_type=jnp.float32)


def _xtx(x):
    rows = x.shape[0]
    blk = 4000
    grid = rows // blk
    return pl.pallas_call(
        _xtx_body,
        grid=(grid,),
        in_specs=[pl.BlockSpec((blk, 128), lambda i: (i, 0))],
        out_specs=pl.BlockSpec((128, 128), lambda i: (0, 0)),
        out_shape=jax.ShapeDtypeStruct((128, 128), jnp.float32),
    )(x)


def _finish_body(e_edges, n_nodes, de, g2_ref, dc2_ref, rc2_ref, atom_ref,
                 xtx_ref, w1_ref, b1_ref, g1_ref, be1_ref, w2_ref, b2_ref,
                 g2v_ref, be2_ref, out_ref, acc_ref, accd_ref):
    ef = jnp.float32(e_edges)
    p = pl.program_id(0)
    i = pl.program_id(1)
    G = g2_ref[0] + g2_ref[1]
    DC = dc2_ref[0] + dc2_ref[1]
    cnt = DC[:, de:de + 1]
    atom = atom_ref[...]
    d = atom.shape[1]
    W1 = w1_ref[...]
    W1a = W1[:, :d]
    W1b = W1[:, d:]
    b1 = b1_ref[...]
    safe = cnt > 0.5
    node = _dotT(jnp.where(safe, atom, 0.0), w2_ref[...]) + b2_ref[...]

    @pl.when(p == 0)
    def _accumulate():
        RC = rc2_ref[0] + rc2_ref[1]
        c = RC[:, de:de + 1]
        R = RC[:, :de]
        z = _dotT(atom, W1a)
        tR = _dotT(R, W1b)
        s = lambda x: jnp.sum(x, axis=0, keepdims=True)
        blk = jnp.concatenate(
            [s(G), s(c * z * z), s(z * tR), s(c * z), s(node),
             s(node * node)], axis=0)

        @pl.when(i == 0)
        def _():
            acc_ref[...] = blk
            accd_ref[...] = s(DC)

        @pl.when(i != 0)
        def _():
            acc_ref[...] += blk
            accd_ref[...] += s(DC)

    @pl.when(p == 1)
    def _apply():
        g1 = g1_ref[...]
        be1 = be1_ref[...]
        g2v = g2v_ref[...]
        be2 = be2_ref[...]
        Gs = acc_ref[0:1, :]
        Ds = accd_ref[:, :de]
        m1 = (_dotT(Gs, W1a) + _dotT(Ds, W1b)) / ef + b1

        xtx = xtx_ref[...]
        dd = jnp.zeros((de, de), jnp.float32)
        for k in range(128 // de):
            dd = dd + xtx[k * de:(k + 1) * de, k * de:(k + 1) * de]
        Wdd = jnp.dot(W1b, dd, precision=lax.Precision.HIGHEST,
                      preferred_element_type=jnp.float32)
        tt = jnp.sum(Wdd * W1b, axis=1)[None, :]

        q = (acc_ref[1:2, :]
             + 2.0 * acc_ref[2:3, :]
             + 2.0 * b1 * acc_ref[3:4, :]
             + tt
             + 2.0 * b1 * _dotT(Ds, W1b)
             + ef * b1 * b1)
        v1 = q / ef - m1 * m1
        scale1 = g1 * lax.rsqrt(v1 + 1e-5)

        Dsum = DC[:, :de]
        inv = 1.0 / jnp.maximum(cnt, 1.0)
        M1 = _dotT(G * inv, W1a) + _dotT(Dsum * inv, W1b) + b1
        fea = jnp.where(safe, (M1 - m1) * scale1 + be1, 0.0)

        nf = jnp.float32(n_nodes)
        m2 = acc_ref[4:5, :] / nf
        v2 = acc_ref[5:6, :] / nf - m2 * m2
        nb = (node - m2) * (g2v * lax.rsqrt(v2 + 1e-5)) + be2

        x = nb + fea
        out_ref[...] = jnp.maximum(x, 0.0) + jnp.log(
            1.0 + jnp.exp(-jnp.abs(x)))


def _finish(e_edges, de, g2, dc2, rc2, atom, xtx, W1, b1, g1, be1, W2, b2,
            g2v, be2):
    n, d = atom.shape
    bn = 2000
    nb = n // bn
    full = lambda shape: pl.BlockSpec(shape, lambda p, i: tuple(
        0 for _ in shape))
    blk3 = lambda w: pl.BlockSpec((2, bn, w), lambda p, i: (0, i, 0))
    return pl.pallas_call(
        functools.partial(_finish_body, e_edges, n, de),
        grid=(2, nb),
        in_specs=[
            blk3(d), blk3(de + 1), blk3(de + 1),
            pl.BlockSpec((bn, d), lambda p, i: (i, 0)),
            full((128, 128)), full((d, d + de)), full((1, d)), full((1, d)),
            full((1, d)), full((d, d)), full((1, d)), full((1, d)),
            full((1, d)),
        ],
        out_specs=pl.BlockSpec((bn, d), lambda p, i: (i, 0)),
        out_shape=jax.ShapeDtypeStruct((n, d), jnp.float32),
        scratch_shapes=[pltpu.VMEM((6, d), jnp.float32),
                        pltpu.VMEM((1, de + 1), jnp.float32)],
    )(g2, dc2, rc2, atom, xtx, W1, b1, g1, be1, W2, b2, g2v, be2)


def kernel(atom_fea, nbr_dist_fea, nbr_adj_value, nbr_bond_type,
           self_fea_idx, nbr_fea_idx, ads_atom_idx,
           W1, b1, g1, be1, W2, b2, g2, be2):
    n, d = atom_fea.shape
    e, de = nbr_dist_fea.shape
    de1 = de + 1

    nw = _NC * _NS
    npad = _pad_n(n)
    d17 = jnp.concatenate(
        [nbr_dist_fea, jnp.ones((e, 1), jnp.float32)], axis=1)
    sidx2 = self_fea_idx.reshape(e // _K, _K)
    nidx2 = nbr_fea_idx.reshape(e // _K, _K)
    zd = jnp.zeros((npad, d), jnp.float32)
    zde1 = jnp.zeros((npad, de1), jnp.float32)

    g2p, dc2p, rc2p = _sc_accumulate(n, e, d, de1)(
        atom_fea, sidx2, nidx2, d17, zd, zde1)

    xtx = _xtx(nbr_dist_fea.reshape(e // 8, 8 * de))

    row = lambda v: v.reshape(1, d)
    return _finish(e, de, g2p, dc2p, rc2p, atom_fea, xtx, W1, row(b1),
                   row(g1), row(be1), W2, row(b2), row(g2), row(be2))


# speedup vs baseline: 5.3145x; 1.1602x over previous
import functools

import jax
import jax.numpy as jnp
from jax import lax
from jax.experimental import pallas as pl
from jax.experimental.pallas import tpu as pltpu
from jax.experimental.pallas import tpu_sc as plsc

_NC = 2
_NS = 16
_K = 125
_INNER = 20


def _pad_n(n):
    return ((n + _NS - 1) // _NS) * _NS


def _dotT(a, b):
    return lax.dot_general(a, b, (((1,), (1,)), ((), ())),
                           precision=lax.Precision.HIGHEST,
                           preferred_element_type=jnp.float32)


@functools.lru_cache(maxsize=None)
def _sc_accumulate(n, e, d, de1):
    nw = _NC * _NS
    ew = e // nw
    npad = _pad_n(n)
    rows_per_tile = npad // _NS
    outer = ew // (_INNER * _K)
    idx_rows_per_worker = ew // _K

    mesh = plsc.VectorSubcoreMesh(core_axis_name="c", subcore_axis_name="s")

    @functools.partial(
        pl.kernel,
        mesh=mesh,
        compiler_params=pltpu.CompilerParams(use_tc_tiling_on_sc=False),
        out_type=[
            jax.ShapeDtypeStruct((_NC, npad, d), jnp.float32),
            jax.ShapeDtypeStruct((_NC, npad, de1), jnp.float32),
            jax.ShapeDtypeStruct((_NC, npad, de1), jnp.float32),
        ],
        scratch_types=[
            pltpu.VMEM((_K, d), jnp.float32),
            pltpu.VMEM((_INNER, _K), jnp.int32),
            pltpu.VMEM((_INNER, _K), jnp.int32),
            pltpu.VMEM((_K, de1), jnp.float32),
            pltpu.VMEM_SHARED((npad, d), jnp.float32),
            pltpu.VMEM_SHARED((npad, de1), jnp.float32),
            pltpu.VMEM_SHARED((npad, de1), jnp.float32),
            pltpu.SemaphoreType.DMA,
            pltpu.SemaphoreType.DMA,
            pltpu.SemaphoreType.DMA,
            pltpu.SemaphoreType.DMA,
            pltpu.SemaphoreType.DMA,
        ],
    )
    def sc_kernel(atom_hbm, sidx_hbm, nidx_hbm, d17_hbm, zd_hbm, zde1_hbm,
                  g2_out, dc2_out, rc2_out,
                  rows_v, sidx_v, nidx_v, d17_v, gacc, dacc, racc,
                  gsem, dsem, s1sem, s2sem, s3sem):
        cid = lax.axis_index("c")
        sid = lax.axis_index("s")
        wid = sid * _NC + cid

        r0 = sid * rows_per_tile
        pltpu.sync_copy(zd_hbm.at[pl.ds(r0, rows_per_tile), :],
                        gacc.at[pl.ds(r0, rows_per_tile), :])
        pltpu.sync_copy(zde1_hbm.at[pl.ds(r0, rows_per_tile), :],
                        dacc.at[pl.ds(r0, rows_per_tile), :])
        pltpu.sync_copy(zde1_hbm.at[pl.ds(r0, rows_per_tile), :],
                        racc.at[pl.ds(r0, rows_per_tile), :])
        plsc.subcore_barrier()

        base_row = wid * idx_rows_per_worker
        edge_base = wid * ew

        def body(o, carry):
            pltpu.sync_copy(sidx_hbm.at[pl.ds(base_row + o * _INNER,
                                              _INNER), :], sidx_v)
            pltpu.sync_copy(nidx_hbm.at[pl.ds(base_row + o * _INNER,
                                              _INNER), :], nidx_v)
            for j in range(_INNER):
                gcp = pltpu.async_copy(atom_hbm.at[nidx_v.at[j]], rows_v,
                                       gsem)
                dcp = pltpu.async_copy(
                    d17_hbm.at[pl.ds(edge_base + (o * _INNER + j) * _K,
                                     _K), :], d17_v, dsem)
                gcp.wait()
                s1 = pltpu.async_copy(rows_v, gacc.at[sidx_v.at[j]], s1sem,
                                      add=True)
                dcp.wait()
                s2 = pltpu.async_copy(d17_v, dacc.at[sidx_v.at[j]], s2sem,
                                      add=True)
                s3 = pltpu.async_copy(d17_v, racc.at[nidx_v.at[j]], s3sem,
                                      add=True)
                s1.wait()
                s2.wait()
                s3.wait()
            return carry

        lax.fori_loop(0, outer, body, 0)
        plsc.subcore_barrier()

        pltpu.sync_copy(gacc.at[pl.ds(r0, rows_per_tile), :],
                        g2_out.at[cid, pl.ds(r0, rows_per_tile), :])
        pltpu.sync_copy(dacc.at[pl.ds(r0, rows_per_tile), :],
                        dc2_out.at[cid, pl.ds(r0, rows_per_tile), :])
        pltpu.sync_copy(racc.at[pl.ds(r0, rows_per_tile), :],
                        rc2_out.at[cid, pl.ds(r0, rows_per_tile), :])

    return sc_kernel


def _xtx_body(x_ref, o_ref):
    @pl.when(pl.program_id(0) == 0)
    def _init():
        o_ref[...] = jnp.zeros_like(o_ref)

    x = x_ref[...]
    o_ref[...] += lax.dot_general(x, x, (((0,), (0,)), ((), ())),
                                  precision=lax.Precision.HIGHEST,
                                  preferred_element_type=jnp.float32)


def _xtx(x):
    rows = x.shape[0]
    blk = 4000
    grid = rows // blk
    return pl.pallas_call(
        _xtx_body,
        grid=(grid,),
        in_specs=[pl.BlockSpec((blk, 128), lambda i: (i, 0))],
        out_specs=pl.BlockSpec((128, 128), lambda i: (0, 0)),
        out_shape=jax.ShapeDtypeStruct((128, 128), jnp.float32),
    )(x)


def _finish_body(e_edges, n_nodes, de, g2_ref, dc2_ref, rc2_ref, atom_ref,
                 xtx_ref, w1_ref, b1_ref, g1_ref, be1_ref, w2_ref, b2_ref,
                 g2v_ref, be2_ref, out_ref, acc_ref, accd_ref):
    ef = jnp.float32(e_edges)
    p = pl.program_id(0)
    i = pl.program_id(1)
    G = g2_ref[0] + g2_ref[1]
    DC = dc2_ref[0] + dc2_ref[1]
    cnt = DC[:, de:de + 1]
    atom = atom_ref[...]
    d = atom.shape[1]
    W1 = w1_ref[...]
    W1a = W1[:, :d]
    W1b = W1[:, d:]
    b1 = b1_ref[...]
    safe = cnt > 0.5
    node = _dotT(jnp.where(safe, atom, 0.0), w2_ref[...]) + b2_ref[...]

    @pl.when(p == 0)
    def _accumulate():
        RC = rc2_ref[0] + rc2_ref[1]
        c = RC[:, de:de + 1]
        R = RC[:, :de]
        z = _dotT(atom, W1a)
        tR = _dotT(R, W1b)
        s = lambda x: jnp.sum(x, axis=0, keepdims=True)
        blk = jnp.concatenate(
            [s(G), s(c * z * z), s(z * tR), s(c * z), s(node),
             s(node * node)], axis=0)

        @pl.when(i == 0)
        def _():
            acc_ref[...] = blk
            accd_ref[...] = s(DC)

        @pl.when(i != 0)
        def _():
            acc_ref[...] += blk
            accd_ref[...] += s(DC)

    @pl.when(p == 1)
    def _apply():
        g1 = g1_ref[...]
        be1 = be1_ref[...]
        g2v = g2v_ref[...]
        be2 = be2_ref[...]
        Gs = acc_ref[0:1, :]
        Ds = accd_ref[:, :de]
        m1 = (_dotT(Gs, W1a) + _dotT(Ds, W1b)) / ef + b1

        xtx = xtx_ref[...]
        dd = jnp.zeros((de, de), jnp.float32)
        for k in range(128 // de):
            dd = dd + xtx[k * de:(k + 1) * de, k * de:(k + 1) * de]
        Wdd = jnp.dot(W1b, dd, precision=lax.Precision.HIGHEST,
                      preferred_element_type=jnp.float32)
        tt = jnp.sum(Wdd * W1b, axis=1)[None, :]

        q = (acc_ref[1:2, :]
             + 2.0 * acc_ref[2:3, :]
             + 2.0 * b1 * acc_ref[3:4, :]
             + tt
             + 2.0 * b1 * _dotT(Ds, W1b)
             + ef * b1 * b1)
        v1 = q / ef - m1 * m1
        scale1 = g1 * lax.rsqrt(v1 + 1e-5)

        Dsum = DC[:, :de]
        inv = 1.0 / jnp.maximum(cnt, 1.0)
        M1 = _dotT(G * inv, W1a) + _dotT(Dsum * inv, W1b) + b1
        fea = jnp.where(safe, (M1 - m1) * scale1 + be1, 0.0)

        nf = jnp.float32(n_nodes)
        m2 = acc_ref[4:5, :] / nf
        v2 = acc_ref[5:6, :] / nf - m2 * m2
        nb = (node - m2) * (g2v * lax.rsqrt(v2 + 1e-5)) + be2

        x = nb + fea
        out_ref[...] = jnp.maximum(x, 0.0) + jnp.log(
            1.0 + jnp.exp(-jnp.abs(x)))


def _finish(e_edges, de, g2, dc2, rc2, atom, xtx, W1, b1, g1, be1, W2, b2,
            g2v, be2):
    n, d = atom.shape
    bn = 2000
    nb = n // bn
    full = lambda shape: pl.BlockSpec(shape, lambda p, i: tuple(
        0 for _ in shape))
    blk3 = lambda w: pl.BlockSpec((2, bn, w), lambda p, i: (0, i, 0))
    return pl.pallas_call(
        functools.partial(_finish_body, e_edges, n, de),
        grid=(2, nb),
        in_specs=[
            blk3(d), blk3(de + 1), blk3(de + 1),
            pl.BlockSpec((bn, d), lambda p, i: (i, 0)),
            full((128, 128)), full((d, d + de)), full((1, d)), full((1, d)),
            full((1, d)), full((d, d)), full((1, d)), full((1, d)),
            full((1, d)),
        ],
        out_specs=pl.BlockSpec((bn, d), lambda p, i: (i, 0)),
        out_shape=jax.ShapeDtypeStruct((n, d), jnp.float32),
        scratch_shapes=[pltpu.VMEM((6, d), jnp.float32),
                        pltpu.VMEM((1, de + 1), jnp.float32)],
    )(g2, dc2, rc2, atom, xtx, W1, b1, g1, be1, W2, b2, g2v, be2)


def kernel(atom_fea, nbr_dist_fea, nbr_adj_value, nbr_bond_type,
           self_fea_idx, nbr_fea_idx, ads_atom_idx,
           W1, b1, g1, be1, W2, b2, g2, be2):
    n, d = atom_fea.shape
    e, de = nbr_dist_fea.shape
    de1 = de + 1

    nw = _NC * _NS
    npad = _pad_n(n)
    d17 = jnp.concatenate(
        [nbr_dist_fea, jnp.ones((e, 1), jnp.float32)], axis=1)
    sidx2 = self_fea_idx.reshape(e // _K, _K)
    nidx2 = nbr_fea_idx.reshape(e // _K, _K)
    zd = jnp.zeros((npad, d), jnp.float32)
    zde1 = jnp.zeros((npad, de1), jnp.float32)

    g2p, dc2p, rc2p = _sc_accumulate(n, e, d, de1)(
        atom_fea, sidx2, nidx2, d17, zd, zde1)

    xtx = _xtx(nbr_dist_fea.reshape(e // 8, 8 * de))

    row = lambda v: v.reshape(1, d)
    return _finish(e, de, g2p, dc2p, rc2p, atom_fea, xtx, W1, row(b1),
                   row(g1), row(be1), W2, row(b2), row(g2), row(be2))
